# baseline (speedup 1.0000x reference)
import functools

import jax
import jax.numpy as jnp
from jax import lax
from jax.experimental import pallas as pl
from jax.experimental.pallas import tpu as pltpu
from jax.experimental.pallas import tpu_sc as plsc

_NC = 2
_NS = 16
_NW = _NC * _NS
_LANES = 16
_W = 128
_NBUF = 4
_WCHUNK = 4


def kernel(datasets, perm):
    B, N, D = datasets.shape
    data2 = datasets.reshape(B, N // 2, 2 * D)
    cpb = N // _W
    perm2d = perm.astype(jnp.int32).reshape(cpb, _W)
    nb_per_w = B // _NW
    m = nb_per_w * cpb

    mesh = plsc.VectorSubcoreMesh(core_axis_name="c", subcore_axis_name="s")

    @functools.partial(
        pl.kernel,
        out_type=jax.ShapeDtypeStruct((B, N // 2, 2 * D), datasets.dtype),
        mesh=mesh,
        scratch_types=[
            pltpu.VMEM((cpb, _W), jnp.int32),
            pltpu.VMEM((cpb, _W), jnp.int32),
            pltpu.VMEM((cpb, _W), jnp.int32),
            pltpu.VMEM((_NBUF, _W, 2 * D), jnp.float32),
            pltpu.VMEM((_NBUF, _W // 2, 2 * D), jnp.float32),
            pltpu.SemaphoreType.DMA((_NBUF,)),
            pltpu.SemaphoreType.DMA((_NBUF,)),
        ],
    )
    def _k(data_hbm, perm_hbm, out_hbm, perm_v, qidx_v, roff_v,
           pair_v, win_v, gsem, wsem):
        wid = lax.axis_index("s") * _NC + lax.axis_index("c")
        pltpu.sync_copy(perm_hbm, perm_v)
        b0 = wid * nb_per_w

        for j in range(cpb):
            for k in range(_W // _LANES):
                sl = pl.ds(k * _LANES, _LANES)
                pv = perm_v[j, sl]
                qidx_v[j, sl] = lax.shift_right_logical(pv, 1)
                roff_v[j, sl] = lax.shift_left(
                    lax.bitwise_and(pv, 1), 6)

        def g_copy(c, s):
            b = b0 + c // cpb
            j = c % cpb
            return pltpu.async_copy(
                data_hbm.at[b].at[qidx_v.at[j]], pair_v.at[s], gsem.at[s])

        def w_copy(c, s):
            b = b0 + c // cpb
            j = c % cpb
            return pltpu.async_copy(
                win_v.at[s],
                out_hbm.at[b].at[pl.ds(j * (_W // 2), _W // 2)],
                wsem.at[s])

        def compact(c, s):
            j = c % cpb
            for g in range(_W // _LANES):
                rv = roff_v[j, pl.ds(g * _LANES, _LANES)]
                for k in range(_LANES):
                    i = g * _LANES + k
                    off = rv[k]
                    for gg in range(D // _LANES):
                        win_v[s, i // 2,
                              pl.ds((i % 2) * D + gg * _LANES, _LANES)] = (
                            pair_v[s, i, pl.ds(off + gg * _LANES, _LANES)])

        @pl.loop(0, m // _WCHUNK)
        def _chunk(q):
            c0 = q * _WCHUNK
            gh = [None] * _WCHUNK
            wh = [None] * _WCHUNK
            gh[0] = g_copy(c0 + 0, 0)
            gh[1] = g_copy(c0 + 1, 1)
            for p in range(_WCHUNK):
                gh[p].wait()
                pn = p + 2
                if pn < _WCHUNK:
                    gh[pn] = g_copy(c0 + pn, pn)
                compact(c0 + p, p)
                wh[p] = w_copy(c0 + p, p)
            for p in range(_WCHUNK):
                wh[p].wait()

    out = _k(data2, perm2d)
    return out.reshape(B, N, D)
